# baseline (speedup 1.0000x reference)
import functools

import jax
import jax.numpy as jnp
from jax import lax
from jax.experimental import pallas as pl

B, N_IMG, N_TXT, D, Q = 4, 21760, 256, 256, 900
BN = 1280

_HI = lax.Precision.DEFAULT


def _maxsim_body(img_ref, txt_ref, w_ref, p_ref, tmask_ref, imask_ref, out_ref):
    x = img_ref[0]
    w = w_ref[...]
    b = p_ref[0:1, :]
    g = p_ref[1:2, :]
    beta = p_ref[2:3, :]
    x = jnp.dot(x, w, precision=_HI) + b
    mu = jnp.mean(x, axis=-1, keepdims=True)
    var = jnp.mean((x - mu) ** 2, axis=-1, keepdims=True)
    x = (x - mu) / jnp.sqrt(var + 1e-5) * g + beta
    txt = txt_ref[0]
    s = lax.dot_general(x, txt, (((1,), (1,)), ((), ())), precision=_HI)
    s = jnp.where(tmask_ref[0] > 0, s, -1e9)
    m = jnp.max(s, axis=-1)
    m = jnp.where(imask_ref[0, 0] > 0, m, -1e9)
    out_ref[0, 0, :] = m


NBLK = N_IMG // BN


def _max_sim(img_feat, txt_feat, W_proj, b_proj, ln_g, ln_b, txt_mask, img_mask):
    params = jnp.zeros((8, D), jnp.float32)
    params = params.at[0].set(b_proj).at[1].set(ln_g).at[2].set(ln_b)
    tmaskf = txt_mask.astype(jnp.float32).reshape(B, 1, N_TXT)
    imaskf = img_mask.astype(jnp.float32).reshape(B * NBLK, 1, BN)
    out = pl.pallas_call(
        _maxsim_body,
        grid=(B, NBLK),
        in_specs=[
            pl.BlockSpec((1, BN, D), lambda b, n: (b, n, 0)),
            pl.BlockSpec((1, N_TXT, D), lambda b, n: (b, 0, 0)),
            pl.BlockSpec((D, D), lambda b, n: (0, 0)),
            pl.BlockSpec((8, D), lambda b, n: (0, 0)),
            pl.BlockSpec((1, 1, N_TXT), lambda b, n: (b, 0, 0)),
            pl.BlockSpec((1, 1, BN), lambda b, n: (b * NBLK + n, 0, 0)),
        ],
        out_specs=pl.BlockSpec((1, 1, BN), lambda b, n: (b * NBLK + n, 0, 0)),
        out_shape=jax.ShapeDtypeStruct((B * NBLK, 1, BN), jnp.float32),
    )(img_feat, txt_feat, W_proj, params, tmaskf, imaskf)
    return out.reshape(B, N_IMG)


def kernel(img_feat, img_mask, img_coor, img_shapes, txt_feat, txt_mask,
           W_proj, b_proj, ln_g, ln_b, cls_init,
           W1, b1, W2, b2, W3, b3):
    max_sim = _max_sim(img_feat, txt_feat, W_proj, b_proj, ln_g, ln_b,
                       txt_mask, img_mask)
    x = img_feat @ W_proj + b_proj
    mu = jnp.mean(x, axis=-1, keepdims=True)
    var = jnp.mean((x - mu) ** 2, axis=-1, keepdims=True)
    x = (x - mu) / jnp.sqrt(var + 1e-5) * ln_g + ln_b
    enc_cls = jnp.einsum('bnd,btd->bnt', x, txt_feat)
    enc_cls = jnp.where(txt_mask[:, None, :], enc_cls, -1e9)
    enc_cls = jnp.where(img_mask[:, :, None], enc_cls, -1e9)
    _, topk_idx = jax.lax.top_k(max_sim, Q)

    level_sizes = jnp.prod(img_shapes, axis=-1)
    cum_sizes = jnp.cumsum(level_sizes)
    n = img_coor.shape[1]
    level_idx = jnp.searchsorted(cum_sizes, jnp.arange(n, dtype=cum_sizes.dtype), side='right')
    wh_flat = (0.05 * (2.0 ** level_idx.astype(jnp.float32))).astype(jnp.float32)
    wh = jnp.broadcast_to(wh_flat[None, :, None], (img_coor.shape[0], n, 2))
    bbox = jnp.concatenate([img_coor, wh], axis=-1)
    bbox = jnp.clip(bbox, 0.01, 0.99)
    bbox_init = jnp.log(bbox / (1.0 - bbox))

    h = jax.nn.relu(x @ W1 + b1)
    h = jax.nn.relu(h @ W2 + b2)
    bbox_update = h @ W3 + b3
    query_bbox = bbox_init + bbox_update
    enc_topk = jnp.take_along_axis(enc_cls, jnp.broadcast_to(topk_idx[:, :, None], (B, Q, N_TXT)), axis=1)
    bbox_topk = jnp.take_along_axis(query_bbox, jnp.broadcast_to(topk_idx[:, :, None], (B, Q, 4)), axis=1)
    query_cls = jnp.broadcast_to(cls_init[None], (B, Q, D))
    out_mask = jnp.zeros((B, Q), dtype=jnp.bool_)
    att_mask = jnp.zeros((B, Q, Q), dtype=jnp.bool_)
    return (enc_topk, jax.nn.sigmoid(bbox_topk), out_mask, query_cls,
            jax.lax.stop_gradient(bbox_topk), att_mask)


# speedup vs baseline: 1.6432x; 1.6432x over previous
import functools

import jax
import jax.numpy as jnp
from jax import lax
from jax.experimental import pallas as pl
from jax.experimental.pallas import tpu as pltpu
from jax.experimental.pallas import tpu_sc as plsc

B, N_IMG, N_TXT, D, Q = 4, 21760, 256, 256, 900
BN = 1280
NBLK = N_IMG // BN
NR, NL = 170, 128
QP = 1024
_HI = lax.Precision.HIGHEST
_DF = lax.Precision.DEFAULT


def _proj_body(img_ref, w_ref, p_ref, out_ref):
    out_ref[0] = jnp.dot(img_ref[0], w_ref[...], precision=_DF) + p_ref[0:1, :]


def _sim_body(x_ref, mu_ref, var_ref, txt_ref, p_ref, tmask_ref, imask_ref,
              out_ref):
    x = x_ref[0]
    mu = mu_ref[0, 0][:, None]
    var = var_ref[0, 0][:, None]
    x = (x - mu) / jnp.sqrt(var + 1e-5) * p_ref[1:2, :] + p_ref[2:3, :]
    s = lax.dot_general(x, txt_ref[0], (((1,), (1,)), ((), ())), precision=_DF)
    s = jnp.where(tmask_ref[0] > 0, s, -1e9)
    m = jnp.max(s, axis=-1)
    m = jnp.where(imask_ref[0, 0] > 0, m, -1e9)
    out_ref[0, 0, :] = m


def _max_sim(img_feat, txt_feat, W_proj, params, tmaskf, imaskf):
    x0 = pl.pallas_call(
        _proj_body,
        grid=(B, NBLK),
        in_specs=[
            pl.BlockSpec((1, BN, D), lambda b, n: (b, n, 0)),
            pl.BlockSpec((D, D), lambda b, n: (0, 0)),
            pl.BlockSpec((8, D), lambda b, n: (0, 0)),
        ],
        out_specs=pl.BlockSpec((1, BN, D), lambda b, n: (b * NBLK + n, 0, 0)),
        out_shape=jax.ShapeDtypeStruct((B * NBLK, BN, D), jnp.float32),
    )(img_feat, W_proj, params)
    x0r = x0.reshape(B, N_IMG, D)
    mu = jnp.mean(x0r, axis=-1)
    var = jnp.mean((x0r - mu[..., None]) ** 2, axis=-1)
    out = pl.pallas_call(
        _sim_body,
        grid=(B, NBLK),
        in_specs=[
            pl.BlockSpec((1, BN, D), lambda b, n: (b * NBLK + n, 0, 0)),
            pl.BlockSpec((1, 1, BN), lambda b, n: (b * NBLK + n, 0, 0)),
            pl.BlockSpec((1, 1, BN), lambda b, n: (b * NBLK + n, 0, 0)),
            pl.BlockSpec((1, N_TXT, D), lambda b, n: (b, 0, 0)),
            pl.BlockSpec((8, D), lambda b, n: (0, 0)),
            pl.BlockSpec((1, 1, N_TXT), lambda b, n: (b, 0, 0)),
            pl.BlockSpec((1, 1, BN), lambda b, n: (b * NBLK + n, 0, 0)),
        ],
        out_specs=pl.BlockSpec((1, 1, BN), lambda b, n: (b * NBLK + n, 0, 0)),
        out_shape=jax.ShapeDtypeStruct((B * NBLK, 1, BN), jnp.float32),
    )(x0, mu.reshape(B * NBLK, 1, BN), var.reshape(B * NBLK, 1, BN),
      txt_feat, params, tmaskf, imaskf)
    return out.reshape(B, N_IMG)


def _excl_cumsum(m, u_lane, l_row, ones_col):
    within = jnp.dot(m, u_lane, precision=_HI)
    rowsum = jnp.dot(m, ones_col, precision=_HI)
    rowoff = jnp.dot(l_row, rowsum, precision=_HI)
    return rowoff + within


def _select_body(ms_ref, out_ref):
    v = ms_ref[0]
    bits = lax.bitcast_convert_type(v, jnp.int32)
    key = jnp.where(bits >= 0, bits, bits ^ jnp.int32(0x7FFFFFFF))

    cnt_nn = jnp.sum((key >= 0).astype(jnp.float32))
    t0 = jnp.where(cnt_nn >= Q, jnp.int32(0), jnp.int32(-2147483648))

    def step(t, T):
        bit = jnp.int32(30) - t
        cand = T | lax.shift_left(jnp.int32(1), bit)
        cnt = jnp.sum((key >= cand).astype(jnp.float32))
        return jnp.where(cnt >= Q, cand, T)

    T = lax.fori_loop(0, 31, step, t0)

    i0 = lax.broadcasted_iota(jnp.int32, (NL, NL), 0)
    i1 = lax.broadcasted_iota(jnp.int32, (NL, NL), 1)
    u_lane = (i0 < i1).astype(jnp.float32)
    r0 = lax.broadcasted_iota(jnp.int32, (NR, NR), 0)
    r1 = lax.broadcasted_iota(jnp.int32, (NR, NR), 1)
    l_row = (r1 < r0).astype(jnp.float32)
    ones_col = jnp.ones((NL, 1), jnp.float32)

    gt = key > T
    eq = key == T
    cgt = jnp.sum(gt.astype(jnp.float32))
    need = jnp.float32(Q) - cgt
    eqpos = _excl_cumsum(eq.astype(jnp.float32), u_lane, l_row, ones_col)
    sel = gt | (eq & (eqpos < need))
    spos = _excl_cumsum(sel.astype(jnp.float32), u_lane, l_row, ones_col)
    out_ref[0] = jnp.where(sel, spos, -1.0)


def _select(ms3):
    return pl.pallas_call(
        _select_body,
        grid=(B,),
        in_specs=[pl.BlockSpec((1, NR, NL), lambda b: (b, 0, 0))],
        out_specs=pl.BlockSpec((1, NR, NL), lambda b: (b, 0, 0)),
        out_shape=jax.ShapeDtypeStruct((B, NR, NL), jnp.float32),
    )(ms3)


def _compact_body(pos_hbm, val_hbm, cx_hbm, cy_hbm,
                  oidx_hbm, oval_hbm, ocx_hbm, ocy_hbm,
                  pos_v, val_v, cx_v, cy_v, sidx_v, sval_v, scx_v, scy_v):
    wid = lax.axis_index("s") * 2 + lax.axis_index("c")

    @pl.when(wid < B)
    def _():
        pltpu.sync_copy(pos_hbm.at[wid], pos_v)
        pltpu.sync_copy(val_hbm.at[wid], val_v)
        pltpu.sync_copy(cx_hbm.at[wid], cx_v)
        pltpu.sync_copy(cy_hbm.at[wid], cy_v)

        def init(i, c):
            z = jnp.zeros((16,), jnp.float32)
            sidx_v[pl.ds(i * 16, 16)] = z
            sval_v[pl.ds(i * 16, 16)] = jnp.full((16,), -3.0e38, jnp.float32)
            scx_v[pl.ds(i * 16, 16)] = z
            scy_v[pl.ds(i * 16, 16)] = z
            return c

        lax.fori_loop(0, QP // 16, init, 0)

        def step(j, c):
            sl = pl.ds(j * 16, 16)
            pv = pos_v[sl]
            m = pv >= 0.0
            pi = pv.astype(jnp.int32)
            gi = (lax.iota(jnp.int32, 16) + j * 16).astype(jnp.float32)
            plsc.store_scatter(sidx_v, [pi], gi, mask=m)
            plsc.store_scatter(sval_v, [pi], val_v[sl], mask=m)
            plsc.store_scatter(scx_v, [pi], cx_v[sl], mask=m)
            plsc.store_scatter(scy_v, [pi], cy_v[sl], mask=m)
            return c

        lax.fori_loop(0, N_IMG // 16, step, 0)
        pltpu.sync_copy(sidx_v, oidx_hbm.at[wid])
        pltpu.sync_copy(sval_v, oval_hbm.at[wid])
        pltpu.sync_copy(scx_v, ocx_hbm.at[wid])
        pltpu.sync_copy(scy_v, ocy_hbm.at[wid])


@functools.cache
def _compact_kernel():
    return pl.kernel(
        _compact_body,
        out_type=tuple(jax.ShapeDtypeStruct((B, QP), jnp.float32)
                       for _ in range(4)),
        mesh=plsc.VectorSubcoreMesh(core_axis_name="c", subcore_axis_name="s"),
        compiler_params=pltpu.CompilerParams(needs_layout_passes=False),
        scratch_types=[pltpu.VMEM((N_IMG,), jnp.float32)] * 4
                      + [pltpu.VMEM((QP,), jnp.float32)] * 4,
    )


def _compact(posneg, ms, coorx, coory):
    return _compact_kernel()(posneg, ms, coorx, coory)


def _rank_body(sidx_ref, sval_ref, scx_ref, scy_ref,
               oflat_ref, oidx_ref, oval_ref, ocx_ref, ocy_ref):
    vrow = sval_ref[0]
    irow = sidx_ref[0]
    e0 = lax.broadcasted_iota(jnp.int32, (QP, QP), 0)
    e1 = lax.broadcasted_iota(jnp.int32, (QP, QP), 1)
    ident = (e0 == e1).astype(jnp.float32)
    cdims = (((1,), (1,)), ((), ()))
    vcol = lax.dot_general(ident, vrow, cdims, precision=_HI)
    icol = lax.dot_general(ident, irow, cdims, precision=_HI)
    beat = ((vcol > vrow) | ((vcol == vrow) & (icol < irow))).astype(jnp.float32)
    ones_row = jnp.ones((1, QP), jnp.float32)
    r = lax.dot_general(ones_row, beat, (((1,), (0,)), ((), ())), precision=_HI)
    qcol = lax.broadcasted_iota(jnp.int32, (QP, 1), 0).astype(jnp.float32)
    perm = (r == qcol).astype(jnp.float32)
    pdims = (((1,), (0,)), ((), ()))
    oidxcol = lax.dot_general(perm, icol, pdims, precision=_HI)
    ovalcol = lax.dot_general(perm, vcol, pdims, precision=_HI)
    cxcol = lax.dot_general(ident, scx_ref[0], cdims, precision=_HI)
    cycol = lax.dot_general(ident, scy_ref[0], cdims, precision=_HI)
    ocx_ref[0] = lax.dot_general(perm, cxcol, pdims, precision=_HI)
    ocy_ref[0] = lax.dot_general(perm, cycol, pdims, precision=_HI)
    oidxcol = jnp.clip(oidxcol, 0.0, float(N_IMG - 1))
    orow = lax.dot_general(oidxcol, ident, (((0,), (0,)), ((), ())), precision=_HI)
    b = pl.program_id(0)
    oflat_ref[0, 0, :] = orow[0].astype(jnp.int32) + b * N_IMG
    oidx_ref[0] = oidxcol.astype(jnp.int32)
    oval_ref[0] = ovalcol


def _rank(sidx3, sval3, scx3, scy3):
    return pl.pallas_call(
        _rank_body,
        grid=(B,),
        in_specs=[pl.BlockSpec((1, 1, QP), lambda b: (b, 0, 0))] * 4,
        out_specs=[
            pl.BlockSpec((1, 1, QP), lambda b: (b, 0, 0)),
            pl.BlockSpec((1, QP, 1), lambda b: (b, 0, 0)),
            pl.BlockSpec((1, QP, 1), lambda b: (b, 0, 0)),
            pl.BlockSpec((1, QP, 1), lambda b: (b, 0, 0)),
            pl.BlockSpec((1, QP, 1), lambda b: (b, 0, 0)),
        ],
        out_shape=[
            jax.ShapeDtypeStruct((B, 1, QP), jnp.int32),
            jax.ShapeDtypeStruct((B, QP, 1), jnp.int32),
            jax.ShapeDtypeStruct((B, QP, 1), jnp.float32),
            jax.ShapeDtypeStruct((B, QP, 1), jnp.float32),
            jax.ShapeDtypeStruct((B, QP, 1), jnp.float32),
        ],
    )(sidx3, sval3, scx3, scy3)


_ROWS_PER_W = (B * QP) // 32


def _gather_body(feat_hbm, flat_hbm, ofeat_hbm, idx_v, feat_v, sem1):
    wid = lax.axis_index("s") * 2 + lax.axis_index("c")
    base = wid * _ROWS_PER_W
    pltpu.sync_copy(flat_hbm.at[pl.ds(base, _ROWS_PER_W)], idx_v)
    pltpu.async_copy(feat_hbm.at[idx_v], feat_v, sem1).wait()
    pltpu.sync_copy(feat_v, ofeat_hbm.at[pl.ds(base, _ROWS_PER_W)])


@functools.cache
def _gather_kernel():
    return pl.kernel(
        _gather_body,
        out_type=jax.ShapeDtypeStruct((B * QP, D), jnp.float32),
        mesh=plsc.VectorSubcoreMesh(core_axis_name="c", subcore_axis_name="s"),
        scratch_types=[
            pltpu.VMEM((_ROWS_PER_W,), jnp.int32),
            pltpu.VMEM((_ROWS_PER_W, D), jnp.float32),
            pltpu.SemaphoreType.DMA,
        ],
    )


def _gather(feat2d, flat):
    return _gather_kernel()(feat2d, flat)


def _final_body(feat_ref, oidx_ref, oval_ref, ocx_ref, ocy_ref, txt_ref,
                tmask_ref, wp_ref, w1_ref, w2_ref, w3_ref, p_ref,
                enc_ref, sig_ref, raw_ref):
    f = feat_ref[0]
    x = jnp.dot(f, wp_ref[...], precision=_DF) + p_ref[0:1, :]
    mu = jnp.mean(x, axis=-1, keepdims=True)
    var = jnp.mean((x - mu) ** 2, axis=-1, keepdims=True)
    x = (x - mu) / jnp.sqrt(var + 1e-5) * p_ref[1:2, :] + p_ref[2:3, :]
    s = lax.dot_general(x, txt_ref[0], (((1,), (1,)), ((), ())), precision=_DF)
    s = jnp.where(tmask_ref[0] > 0, s, -1e9)
    bad = oval_ref[0] == -1e9
    enc_ref[0] = jnp.where(bad, -1e9, s)

    h = jax.nn.relu(jnp.dot(x, w1_ref[...], precision=_DF) + p_ref[3:4, :])
    h = jax.nn.relu(jnp.dot(h, w2_ref[...], precision=_DF) + p_ref[4:5, :])
    upd = jnp.dot(h, w3_ref[...], precision=_DF) + p_ref[5:6, 0:4]

    idxf = oidx_ref[0].astype(jnp.float32)
    lvl = ((idxf >= p_ref[6:7, 0:1]).astype(jnp.float32)
           + (idxf >= p_ref[6:7, 1:2]).astype(jnp.float32)
           + (idxf >= p_ref[6:7, 2:3]).astype(jnp.float32))
    two_lvl = jnp.where(lvl == 0, 1.0,
                        jnp.where(lvl == 1, 2.0,
                                  jnp.where(lvl == 2, 4.0, 8.0)))
    wh = jnp.float32(0.05) * two_lvl
    cx = ocx_ref[0]
    cy = ocy_ref[0]
    ci = lax.broadcasted_iota(jnp.int32, (QP, 4), 1)
    bb = jnp.where(ci == 0, cx, jnp.where(ci == 1, cy, wh))
    bb = jnp.clip(bb, 0.01, 0.99)
    q = jnp.log(bb / (1.0 - bb)) + upd
    sig_ref[0] = jax.nn.sigmoid(q)
    raw_ref[0] = q


def _final(selfeat, oidxc, ovalc, ocxc, ocyc, txt_feat, tmaskf,
           W_proj, W1, W2, W3, params):
    return pl.pallas_call(
        _final_body,
        grid=(B,),
        in_specs=[
            pl.BlockSpec((1, QP, D), lambda b: (b, 0, 0)),
            pl.BlockSpec((1, QP, 1), lambda b: (b, 0, 0)),
            pl.BlockSpec((1, QP, 1), lambda b: (b, 0, 0)),
            pl.BlockSpec((1, QP, 1), lambda b: (b, 0, 0)),
            pl.BlockSpec((1, QP, 1), lambda b: (b, 0, 0)),
            pl.BlockSpec((1, N_TXT, D), lambda b: (b, 0, 0)),
            pl.BlockSpec((1, 1, N_TXT), lambda b: (b, 0, 0)),
            pl.BlockSpec((D, D), lambda b: (0, 0)),
            pl.BlockSpec((D, D), lambda b: (0, 0)),
            pl.BlockSpec((D, D), lambda b: (0, 0)),
            pl.BlockSpec((D, 4), lambda b: (0, 0)),
            pl.BlockSpec((8, D), lambda b: (0, 0)),
        ],
        out_specs=[
            pl.BlockSpec((1, QP, N_TXT), lambda b: (b, 0, 0)),
            pl.BlockSpec((1, QP, 4), lambda b: (b, 0, 0)),
            pl.BlockSpec((1, QP, 4), lambda b: (b, 0, 0)),
        ],
        out_shape=[
            jax.ShapeDtypeStruct((B, QP, N_TXT), jnp.float32),
            jax.ShapeDtypeStruct((B, QP, 4), jnp.float32),
            jax.ShapeDtypeStruct((B, QP, 4), jnp.float32),
        ],
    )(selfeat, oidxc, ovalc, ocxc, ocyc, txt_feat, tmaskf, W_proj, W1, W2, W3, params)


def kernel(img_feat, img_mask, img_coor, img_shapes, txt_feat, txt_mask,
           W_proj, b_proj, ln_g, ln_b, cls_init,
           W1, b1, W2, b2, W3, b3):
    params = jnp.zeros((8, D), jnp.float32)
    params = (params.at[0].set(b_proj).at[1].set(ln_g).at[2].set(ln_b)
              .at[3].set(b1).at[4].set(b2)
              .at[5, 0:4].set(b3))
    cums = jnp.cumsum(jnp.prod(img_shapes, axis=-1)).astype(jnp.float32)
    params = params.at[6, 0:3].set(cums[0:3])
    tmaskf = txt_mask.astype(jnp.float32).reshape(B, 1, N_TXT)
    imaskf = img_mask.astype(jnp.float32).reshape(B * NBLK, 1, BN)

    ms = _max_sim(img_feat, txt_feat, W_proj, params, tmaskf, imaskf)
    posneg = _select(ms.reshape(B, NR, NL))
    coorx = img_coor[:, :, 0]
    coory = img_coor[:, :, 1]
    sidx, sval, scx, scy = _compact(posneg.reshape(B, N_IMG), ms, coorx, coory)
    oflat, oidxc, ovalc, ocxc, ocyc = _rank(
        sidx.reshape(B, 1, QP), sval.reshape(B, 1, QP),
        scx.reshape(B, 1, QP), scy.reshape(B, 1, QP))
    selfeat = _gather(img_feat.reshape(B * N_IMG, D), oflat.reshape(B * QP))
    enc, sig, raw = _final(selfeat.reshape(B, QP, D),
                           oidxc, ovalc, ocxc, ocyc, txt_feat, tmaskf,
                           W_proj, W1, W2, W3, params)

    query_cls = jnp.broadcast_to(cls_init[None], (B, Q, D))
    out_mask = jnp.zeros((B, Q), dtype=jnp.bool_)
    att_mask = jnp.zeros((B, Q, Q), dtype=jnp.bool_)
    return (enc[:, :Q], sig[:, :Q], out_mask, query_cls,
            lax.stop_gradient(raw[:, :Q]), att_mask)
